# baseline (speedup 1.0000x reference)
import functools

import jax
import jax.numpy as jnp
from jax import lax
from jax.experimental import pallas as pl
from jax.experimental.pallas import tpu as pltpu
from jax.experimental.pallas import tpu_sc as plsc

C = 256
NQ = C // 128
NPW = 512
SH = 9
NEG = -3.0e38
POS = 3.0e38

_MESH = dict(core_axis_name="c", subcore_axis_name="s",
             num_cores=2, num_subcores=16)
_CPARAMS = dict(needs_layout_passes=False, use_tc_tiling_on_sc=False)


def _make_bin_kernels(n, E, E_in_alloc, E_alloc, NWIN):
    NW = 32
    E50 = E // NW
    NCH = (E50 + C - 1) // C
    mesh = plsc.VectorSubcoreMesh(**_MESH)

    @functools.partial(
        pl.kernel,
        out_type=jax.ShapeDtypeStruct((NW * 256,), jnp.int32),
        mesh=mesh,
        compiler_params=pltpu.CompilerParams(**_CPARAMS),
        scratch_types=[
            pltpu.VMEM((C,), jnp.int32),
            pltpu.VMEM((256,), jnp.int32),
        ],
    )
    def b1(dst_h, hist_h, dv, histv):
        w = lax.axis_index("c") * 16 + lax.axis_index("s")
        lanes = lax.iota(jnp.int32, 16)
        zero16 = jnp.zeros((16,), jnp.int32)
        one16 = jnp.ones((16,), jnp.int32)
        for g in range(16):
            histv[pl.ds(g * 16, 16)] = zero16
        base = w * E50

        def chunk(k, _):
            off = base + k * C
            pltpu.sync_copy(dst_h.at[pl.ds(off, C)], dv)
            rem = E50 - k * C

            def g_body(g, _):
                dvec = dv[pl.ds(g * 16, 16)]
                validv = (jnp.full((16,), g * 16, jnp.int32) + lanes) < rem
                bkt = lax.shift_right_logical(dvec, SH)
                plsc.addupdate_scatter(histv, [bkt], one16, mask=validv)
                return 0

            lax.fori_loop(0, C // 16, g_body, 0)
            return 0

        lax.fori_loop(0, NCH, chunk, 0)
        pltpu.sync_copy(histv, hist_h.at[pl.ds(w * 256, 256)])

    @functools.partial(
        pl.kernel,
        out_type=(jax.ShapeDtypeStruct((E_alloc,), jnp.int32),
                  jax.ShapeDtypeStruct((E_alloc,), jnp.int32),
                  jax.ShapeDtypeStruct((256,), jnp.int32)),
        mesh=mesh,
        compiler_params=pltpu.CompilerParams(**_CPARAMS),
        scratch_types=[
            pltpu.VMEM((NW * 256,), jnp.int32),
            pltpu.VMEM((256,), jnp.int32),
            pltpu.VMEM((256,), jnp.int32),
            pltpu.VMEM((C,), jnp.int32),
            pltpu.VMEM((C,), jnp.int32),
            pltpu.VMEM((64,), jnp.int32),
            pltpu.VMEM((128,), jnp.int32),
            pltpu.VMEM((128,), jnp.int32),
            pltpu.VMEM((128,), jnp.int32),
            pltpu.VMEM((128,), jnp.int32),
            pltpu.VMEM((128,), jnp.int32),
            pltpu.VMEM((128,), jnp.int32),
            pltpu.SemaphoreType.DMA,
            pltpu.SemaphoreType.DMA,
        ],
    )
    def b2(src_h, dst_h, hist_h, bsrc_h, bdst_h, boff_h,
           hall, offv, boffv, sv, dv, tmp, ps0, ps1, ss0, ss1, ds0, ds1,
           sem0, sem1):
        w = lax.axis_index("c") * 16 + lax.axis_index("s")
        lanes = lax.iota(jnp.int32, 16)
        zero16 = jnp.zeros((16,), jnp.int32)
        lane0 = lanes == 0
        lane15 = lanes == 15
        pltpu.sync_copy(hist_h, hall)
        carry = jnp.int32(0)
        for vb in range(16):
            tot = zero16
            my = zero16
            for wp in range(NW):
                t_v = hall[pl.ds(wp * 256 + vb * 16, 16)]
                tot = tot + t_v
                pred = jnp.full((16,), wp < w)
                my = my + jnp.where(pred, t_v, zero16)
            inc = plsc.cumsum(tot)
            excl = inc - tot + carry
            boffv[pl.ds(vb * 16, 16)] = excl
            offv[pl.ds(vb * 16, 16)] = excl + my
            carry = carry + inc[15]

        @pl.when(w == 0)
        def _():
            pltpu.sync_copy(boffv, boff_h)

        base = w * E50
        stg = ((ps0, ss0, ds0, sem0), (ps1, ss1, ds1, sem1))

        def chunk(k, _):
            off = base + k * C
            pltpu.sync_copy(src_h.at[pl.ds(off, C)], sv)
            pltpu.sync_copy(dst_h.at[pl.ds(off, C)], dv)
            rem = E50 - k * C
            for h in range(2):
                ps, ss, dsg, sem = stg[h]

                @pl.when(k > 0)
                def _(ps=ps, ss=ss, dsg=dsg, sem=sem):
                    pltpu.make_async_copy(ss, bsrc_h.at[ps], sem).wait()
                    pltpu.make_async_copy(dsg, bdst_h.at[ps], sem).wait()

                for g in range(8):
                    goff = h * 128 + g * 16
                    svec = sv[pl.ds(goff, 16)]
                    dvec = dv[pl.ds(goff, 16)]
                    validv = (jnp.full((16,), goff, jnp.int32) + lanes) < rem
                    bkt = jnp.where(validv,
                                    lax.shift_right_logical(dvec, SH),
                                    jnp.full((16,), 255, jnp.int32))
                    ks, perm = plsc.sort_key_val(bkt, lanes)
                    tmp[pl.ds(0, 16)] = ks
                    tmp[pl.ds(16, 16)] = svec
                    tmp[pl.ds(32, 16)] = dvec
                    prev = plsc.load_gather(tmp, [jnp.maximum(lanes - 1, 0)])
                    m_st = jnp.logical_or(ks != prev, lane0)
                    runstart = plsc.cummax(
                        jnp.where(m_st, lanes, jnp.full((16,), -1, jnp.int32)))
                    rank = lanes - runstart
                    nxt = plsc.load_gather(
                        tmp, [jnp.minimum(lanes + 1, 15)])
                    m_end = jnp.logical_or(ks != nxt, lane15)
                    base16 = plsc.load_gather(offv, [ks])
                    pos = base16 + rank
                    plsc.store_scatter(offv, [ks], pos + 1, mask=m_end)
                    src_s = plsc.load_gather(tmp, [perm + 16])
                    dst_s = plsc.load_gather(tmp, [perm + 32])
                    ps[pl.ds(g * 16, 16)] = pos
                    ss[pl.ds(g * 16, 16)] = src_s
                    dsg[pl.ds(g * 16, 16)] = dst_s
                pltpu.make_async_copy(ss, bsrc_h.at[ps], sem).start()
                pltpu.make_async_copy(dsg, bdst_h.at[ps], sem).start()
            return 0

        lax.fori_loop(0, NCH, chunk, 0)
        for h in range(2):
            ps, ss, dsg, sem = stg[h]
            pltpu.make_async_copy(ss, bsrc_h.at[ps], sem).wait()
            pltpu.make_async_copy(dsg, bdst_h.at[ps], sem).wait()

    return b1, b2


def _make_edge_stats(F_pad, NWIN, E_alloc, n):
    NV = F_pad // 16
    N_pad = NWIN * NPW
    AW = NPW * F_pad
    NJ = (NWIN + 31) // 32
    mesh = plsc.VectorSubcoreMesh(**_MESH)

    out_t = tuple(jax.ShapeDtypeStruct((N_pad * F_pad,), jnp.float32)
                  for _ in range(4))

    @functools.partial(
        pl.kernel,
        out_type=out_t,
        mesh=mesh,
        compiler_params=pltpu.CompilerParams(**_CPARAMS),
        scratch_types=[
            pltpu.VMEM((AW,), jnp.float32),
            pltpu.VMEM((AW,), jnp.float32),
            pltpu.VMEM((AW,), jnp.float32),
            pltpu.VMEM((AW,), jnp.float32),
            pltpu.VMEM((C,), jnp.int32),
            pltpu.VMEM((C,), jnp.int32),
            pltpu.VMEM((C,), jnp.int32),
            pltpu.VMEM((C,), jnp.int32),
            pltpu.VMEM((C, F_pad), jnp.float32),
            pltpu.VMEM((C, F_pad), jnp.float32),
            pltpu.VMEM((256,), jnp.int32),
            pltpu.SemaphoreType.DMA,
            pltpu.SemaphoreType.DMA,
            pltpu.SemaphoreType.DMA,
            pltpu.SemaphoreType.DMA,
        ],
    )
    def kern(t_h, src_h, dst_h, boff_h, z_h, neg_h, pos_h,
             S_h, Q_h, M_h, N_h,
             accS, accQ, accM, accN, svA, svB, dvA, dvB, rowsA, rowsB,
             boffv, semiA, semiB, semgA, semgB):
        w = lax.axis_index("c") * 16 + lax.axis_index("s")
        pltpu.sync_copy(boff_h, boffv)
        lanes = lax.iota(jnp.int32, 16)
        zero16 = jnp.zeros((16,), jnp.float32)
        neg16 = jnp.full((16,), NEG, jnp.float32)
        pos16 = jnp.full((16,), POS, jnp.float32)
        cnt_lane = lanes == 15
        bufs = ((svA, dvA, rowsA, semiA, semgA),
                (svB, dvB, rowsB, semiB, semgB))

        def win_body(j):
            ev = boffv[pl.ds(j, 16)]
            e0 = ev[0]
            e1 = ev[1]
            basee = (e0 // 8) * 8
            nch = (e1 - basee + (C - 1)) // C
            nch2 = (nch + 1) // 2
            pltpu.sync_copy(z_h, accS)
            pltpu.sync_copy(z_h, accQ)
            pltpu.sync_copy(neg_h, accM)
            pltpu.sync_copy(pos_h, accN)

            def start_idx(k, b):
                sv, dv, _, semi, _ = bufs[b]
                off = basee + k * C
                pltpu.make_async_copy(src_h.at[pl.ds(off, C)], sv, semi).start()
                pltpu.make_async_copy(dst_h.at[pl.ds(off, C)], dv, semi).start()

            def wait_idx(k, b):
                sv, dv, _, semi, _ = bufs[b]
                off = basee + k * C
                pltpu.make_async_copy(src_h.at[pl.ds(off, C)], sv, semi).wait()
                pltpu.make_async_copy(dst_h.at[pl.ds(off, C)], dv, semi).wait()

            def clamp_idx(b):
                sv = bufs[b][0]
                for g in range(C // 16):
                    s = sv[pl.ds(g * 16, 16)]
                    sv[pl.ds(g * 16, 16)] = jnp.clip(s, 0, n - 1)

            def start_gather(b):
                sv, _, rows, _, semg = bufs[b]
                for q in range(NQ):
                    pltpu.make_async_copy(
                        t_h.at[sv.at[pl.ds(q * 128, 128)]],
                        rows.at[pl.ds(q * 128, 128)], semg).start()

            def wait_gather(b):
                sv, _, rows, _, semg = bufs[b]
                for q in range(NQ):
                    pltpu.make_async_copy(
                        t_h.at[sv.at[pl.ds(q * 128, 128)]],
                        rows.at[pl.ds(q * 128, 128)], semg).wait()

            start_idx(0, 0)
            wait_idx(0, 0)
            clamp_idx(0)
            start_gather(0)
            start_idx(1, 1)

            def compute_chunk(k, b):
                _, dv, rows, _, _ = bufs[b]
                off = basee + k * C

                def body16(g, _):
                    goff = g * 16
                    dvec = dv[pl.ds(goff, 16)]
                    for u in range(16):
                        pos = off + goff + u
                        d = dvec[u]
                        valid = jnp.logical_and(pos >= e0, pos < e1)
                        validv = jnp.full((16,), valid)
                        doff = jnp.clip(d - j * NPW, 0, NPW - 1)
                        addr = doff * F_pad
                        validf = jnp.where(
                            cnt_lane,
                            jnp.full((16,), valid.astype(jnp.float32)),
                            zero16)
                        for i in range(NV):
                            sl = pl.ds(addr + 16 * i, 16)
                            vi = rows[goff + u, pl.ds(16 * i, 16)]
                            vS = jnp.where(validv, vi, zero16)
                            if i == NV - 1:
                                plsc.addupdate(accS.at[sl], vS + validf)
                            else:
                                plsc.addupdate(accS.at[sl], vS)
                            plsc.addupdate(accQ.at[sl], vS * vS)
                            vM = jnp.where(validv, vi, neg16)
                            accM[sl] = jnp.maximum(accM[sl], vM)
                            vN = jnp.where(validv, vi, pos16)
                            accN[sl] = jnp.minimum(accN[sl], vN)
                    return 0

                lax.fori_loop(0, C // 16, body16, 0)

            def chunk2_body(k2, _):
                k = k2 * 2
                wait_gather(0)
                wait_idx(k + 1, 1)
                clamp_idx(1)
                start_gather(1)
                compute_chunk(k, 0)
                start_idx(k + 2, 0)
                wait_gather(1)
                wait_idx(k + 2, 0)
                clamp_idx(0)
                start_gather(0)
                compute_chunk(k + 1, 1)
                start_idx(k + 3, 1)
                return 0

            lax.fori_loop(0, nch2, chunk2_body, 0)
            wait_gather(0)
            wait_idx(2 * nch2 + 1, 1)

            pltpu.sync_copy(accS, S_h.at[pl.ds(j * AW, AW)])
            pltpu.sync_copy(accQ, Q_h.at[pl.ds(j * AW, AW)])
            pltpu.sync_copy(accM, M_h.at[pl.ds(j * AW, AW)])
            pltpu.sync_copy(accN, N_h.at[pl.ds(j * AW, AW)])

        def j_loop(jj, _):
            j = w + jj * 32

            @pl.when(j < NWIN)
            def _():
                win_body(j)

            return 0

        lax.fori_loop(0, NJ, j_loop, 0)

    return kern


def _dense_post(x, a, S, Q, Mx, Mn, cnt, has, denom, avg_lin, avg_log,
                postW, postb, linW, linb):
    meanb = S / denom
    mean = jnp.where(has, a + meanb, 0.0)
    mx = jnp.where(has, a + Mx, 0.0)
    mn = jnp.where(has, a + Mn, 0.0)
    var = jnp.where(has, jax.nn.relu(Q / denom - meanb * meanb), 0.0)
    std = jnp.sqrt(var + 1e-5)
    base = jnp.concatenate([mean, mn, mx, std], axis=-1)
    deg = jnp.clip(cnt, 1.0)[:, None]
    s_amp = jnp.log(deg + 1.0) / avg_log
    out = jnp.concatenate(
        [x, base, base * s_amp, base / s_amp, base * (deg / avg_lin)], axis=-1)
    out = out @ postW + postb
    return out @ linW + linb


def kernel(x, edge_index, c1_pre_W, c1_pre_b, c1_post_W, c1_post_b, c1_lin_W, c1_lin_b, c2_pre_W, c2_pre_b, c2_post_W, c2_post_b, c2_lin_W, c2_lin_b, out_W, out_b):
    n = x.shape[0]
    E = edge_index.shape[1]
    NWIN = (n + NPW - 1) // NPW
    E_in_alloc = E + 2 * C
    E_alloc = E + 16384

    src0 = jnp.zeros((E_in_alloc,), jnp.int32).at[:E].set(edge_index[0])
    dst0 = jnp.zeros((E_in_alloc,), jnp.int32).at[:E].set(edge_index[1])

    b1, b2 = _make_bin_kernels(n, E, E_in_alloc, E_alloc, NWIN)
    hist = b1(dst0)
    bsrc, bdst, boff = b2(src0, dst0, hist)

    def run_conv(h, F, F_pad, preW, preb, postW, postb, linW, linb,
                 cnt=None, has=None, denom=None, avg_lin=None, avg_log=None):
        N_pad = NWIN * NPW
        a = h @ preW[:F] + preb
        t = h @ preW[F:]
        t_pad = jnp.zeros((n, F_pad), jnp.float32).at[:, :F].set(t)
        z = jnp.zeros((NPW * F_pad,), jnp.float32)
        ng = jnp.full((NPW * F_pad,), NEG, jnp.float32)
        ps = jnp.full((NPW * F_pad,), POS, jnp.float32)
        kern = _make_edge_stats(F_pad, NWIN, E_alloc, n)
        S, Q, Mx, Mn = kern(t_pad, bsrc, bdst, boff, z, ng, ps)
        S = S.reshape(N_pad, F_pad)
        Q = Q.reshape(N_pad, F_pad)
        Mx = Mx.reshape(N_pad, F_pad)
        Mn = Mn.reshape(N_pad, F_pad)
        if cnt is None:
            cnt = S[:n, F_pad - 1]
            has = (cnt > 0)[:, None]
            denom = jnp.clip(cnt, 1.0)[:, None]
            avg_lin = jnp.mean(cnt)
            avg_log = jnp.mean(jnp.log(cnt + 1.0))
        out = _dense_post(h, a, S[:n, :F], Q[:n, :F], Mx[:n, :F], Mn[:n, :F],
                          cnt, has, denom, avg_lin, avg_log,
                          postW, postb, linW, linb)
        return out, cnt, has, denom, avg_lin, avg_log

    h1, cnt, has, denom, avg_lin, avg_log = run_conv(
        x, 3, 16, c1_pre_W, c1_pre_b, c1_post_W, c1_post_b,
        c1_lin_W, c1_lin_b)
    h1 = jax.nn.relu(h1)
    h2, *_ = run_conv(h1, 20, 32, c2_pre_W, c2_pre_b, c2_post_W,
                      c2_post_b, c2_lin_W, c2_lin_b, cnt, has, denom,
                      avg_lin, avg_log)
    h2 = jax.nn.relu(h2)
    return h2 @ out_W + out_b
